# Initial kernel scaffold; baseline (speedup 1.0000x reference)
import jax
import jax.numpy as jnp
from jax.experimental import pallas as pl


def kernel(graph, r_ij, x_a, x_v, x_d, params):
    raise NotImplementedError("write your pallas kernel here")


# speedup vs baseline: 47.0603x; 47.0603x over previous
import functools

import jax
import jax.numpy as jnp
from jax.experimental import pallas as pl
from jax.experimental.pallas import tpu as pltpu

N_NODES = 50000
N_EDGES = 800000
DIM_A = 16
DIM_V = 8
DIM_D = 4
RANK = 8
R0 = 5.0
NRAD = 8
FW = 80

TP_ORDER = ('tp_000', 'tp_011', 'tp_110', 'tp_101', 'tp_112',
            'tp_220', 'tp_211', 'tp_202', 'tp_222')

_POFF = {'tp_000': 0, 'tp_011': 8, 'tp_110': 16, 'tp_101': 40, 'tp_112': 64,
         'tp_220': 88, 'tp_211': 160, 'tp_202': 232, 'tp_222': 304}
PW = 384


def _kron_eye(w, k):
    return jnp.kron(jnp.eye(k, dtype=w.dtype), w)


def _build_tc_params(params):
    wp = jnp.zeros((FW, PW), jnp.float32)
    wp = wp.at[0:16, 0:8].set(params['tp_000']['Wl'])
    wp = wp.at[0:16, 8:16].set(params['tp_011']['Wl'])
    wp = wp.at[16:40, 16:40].set(_kron_eye(params['tp_110']['Wl'], 3))
    wp = wp.at[16:40, 40:64].set(_kron_eye(params['tp_101']['Wl'], 3))
    wp = wp.at[16:40, 64:88].set(_kron_eye(params['tp_112']['Wl'], 3))
    wp = wp.at[40:76, 88:160].set(_kron_eye(params['tp_220']['Wl'], 9))
    wp = wp.at[40:76, 160:232].set(_kron_eye(params['tp_211']['Wl'], 9))
    wp = wp.at[40:76, 232:304].set(_kron_eye(params['tp_202']['Wl'], 9))
    wp = wp.at[40:76, 304:376].set(_kron_eye(params['tp_222']['Wl'], 9))
    wr = jnp.concatenate([params[t]['Wr'] for t in TP_ORDER], axis=1)
    woA = jnp.concatenate([params['tp_000']['Wo'], params['tp_110']['Wo'],
                           params['tp_220']['Wo']], axis=0)
    return dict(
        wpT=wp.T,
        wrT=wr.T,
        woA=woA.T,
        wo011=params['tp_011']['Wo'].T,
        wo101=params['tp_101']['Wo'].T,
        wo211=params['tp_211']['Wo'].T,
        wo112=params['tp_112']['Wo'].T,
        wo202=params['tp_202']['Wo'].T,
        wo222=params['tp_222']['Wo'].T,
    )


def _tc_body(g_ref, r_ref, wpT, wrT, woA, wo011, wo101, wo211, wo112,
             wo202, wo222, o_ref):
    f32 = jnp.float32
    dn_t = (((1,), (1,)), ((), ()))
    dn_m = (((1,), (0,)), ((), ()))
    dot = functools.partial(jax.lax.dot_general,
                            preferred_element_type=f32,
                            precision=jax.lax.Precision.HIGHEST)
    G = g_ref[...]
    Bsz = G.shape[0]
    r3 = r_ref[...][0:3, :]
    rsq = jnp.sum(r3 * r3, axis=0, keepdims=True) / R0
    srt = jnp.sqrt(rsq)
    npi = jnp.pi * jax.lax.broadcasted_iota(
        jnp.int32, (NRAD, 1), 0).astype(f32)
    env = jnp.maximum(1.0 - rsq, 0.0)
    rad = jnp.cos(npi * srt) * env
    y = r3 * (17.0 / R0)
    nrm = jnp.sqrt(jnp.sum(y * y, axis=0, keepdims=True) + 1e-12)
    rs = y * (jnp.tanh(nrm) / nrm)
    rs_l = [rs[i:i + 1, :] for i in range(3)]

    Gall = dot(wrT[...], rad, dn_m)
    P = dot(wpT[...], G, dn_t)
    g = {t: Gall[k * 8:(k + 1) * 8, :] for k, t in enumerate(TP_ORDER)}

    def pblk(t, k, n=8):
        o = _POFF[t] + k * n
        return P[o:o + n, :]

    h000 = pblk('tp_000', 0) * g['tp_000']
    d110 = sum(pblk('tp_110', i) * rs_l[i] for i in range(3))
    h110 = g['tp_110'] * d110
    d220 = sum(pblk('tp_220', i * 3 + j) * (rs_l[i] * rs_l[j])
               for i in range(3) for j in range(3))
    h220 = g['tp_220'] * d220
    psiA = dot(woA[...], jnp.concatenate([h000, h110, h220], axis=0), dn_m)

    t011 = dot(wo011[...], pblk('tp_011', 0) * g['tp_011'], dn_m)
    psiV_parts = []
    for i in range(3):
        v1 = dot(wo101[...], pblk('tp_101', i) * g['tp_101'], dn_m)
        w = sum(pblk('tp_211', i * 3 + j) * rs_l[j] for j in range(3))
        v2 = dot(wo211[...], g['tp_211'] * w, dn_m)
        psiV_parts.append(t011 * rs_l[i] + v1 + v2)
    psiV = jnp.concatenate(psiV_parts, axis=0)

    D = []
    for i in range(3):
        d1 = dot(wo112[...], pblk('tp_112', i) * g['tp_112'], dn_m)
        w2 = sum(pblk('tp_222', i * 3 + j) * rs_l[j] for j in range(3))
        d2 = dot(wo222[...], g['tp_222'] * w2, dn_m)
        D.append(d1 + d2)
    psiD_parts = []
    for i in range(3):
        for j in range(3):
            e202 = dot(wo202[...], pblk('tp_202', i * 3 + j) * g['tp_202'], dn_m)
            psiD_parts.append(e202 + D[i] * rs_l[j])
    psiD = jnp.concatenate(psiD_parts, axis=0)

    psi = jnp.concatenate(
        [psiA, psiV, psiD, jnp.zeros((4, Bsz), f32)], axis=0)
    o_ref[...] = dot(psi, jnp.eye(FW, dtype=f32),
                     (((0,), (0,)), ((), ())))


def _tc_compute(G, Rpad, tcp, block_e):
    E = G.shape[0]
    grid = (E // block_e,)
    full = lambda a: pl.BlockSpec(a.shape, lambda i: (0,) * a.ndim)
    porder = ('wpT', 'wrT', 'woA', 'wo011', 'wo101', 'wo211', 'wo112',
              'wo202', 'wo222')
    return pl.pallas_call(
        _tc_body,
        grid=grid,
        in_specs=[
            pl.BlockSpec((block_e, FW), lambda i: (i, 0)),
            pl.BlockSpec((8, block_e), lambda i: (0, i)),
        ] + [full(tcp[k]) for k in porder],
        out_specs=pl.BlockSpec((block_e, FW), lambda i: (i, 0)),
        out_shape=jax.ShapeDtypeStruct((E, FW), jnp.float32),
    )(G, Rpad, *[tcp[k] for k in porder])


def _build_table(x_a, x_v, x_d):
    n = x_a.shape[0]
    xv = jnp.transpose(x_v, (0, 2, 1)).reshape(n, 24)
    xd = jnp.transpose(x_d, (0, 2, 3, 1)).reshape(n, 36)
    return jnp.concatenate(
        [x_a, xv, xd, jnp.zeros((n, 4), jnp.float32)], axis=1)


def _unpack_out(out):
    n = out.shape[0]
    B_a = out[:, 0:16]
    B_v = jnp.transpose(out[:, 16:40].reshape(n, 3, 8), (0, 2, 1))
    B_d = jnp.transpose(out[:, 40:76].reshape(n, 3, 3, 4), (0, 3, 1, 2))
    return B_a, B_v, B_d


def kernel(graph, r_ij, x_a, x_v, x_d, params):
    E = r_ij.shape[0]
    n = x_a.shape[0]
    src = graph[0]
    dst = graph[1]
    table = _build_table(x_a, x_v, x_d)
    Rpad = jnp.concatenate(
        [r_ij.T, jnp.zeros((5, E), jnp.float32)], axis=0)
    tcp = _build_tc_params(params)
    block_e = 1280 if E % 1280 == 0 else 16
    G = jnp.take(table, dst, axis=0)
    psi = _tc_compute(G, Rpad, tcp, block_e)
    out = jax.ops.segment_sum(psi, src, num_segments=n)
    return _unpack_out(out)


# speedup vs baseline: 57.6313x; 1.2246x over previous
import functools

import jax
import jax.numpy as jnp
from jax import lax
from jax.experimental import pallas as pl
from jax.experimental.pallas import tpu as pltpu
from jax.experimental.pallas import tpu_sc as plsc

N_NODES = 50000
N_EDGES = 800000
DIM_A = 16
DIM_V = 8
DIM_D = 4
RANK = 8
R0 = 5.0
NRAD = 8
FW = 80

TP_ORDER = ('tp_000', 'tp_011', 'tp_110', 'tp_101', 'tp_112',
            'tp_220', 'tp_211', 'tp_202', 'tp_222')

_POFF = {'tp_000': 0, 'tp_011': 8, 'tp_110': 16, 'tp_101': 40, 'tp_112': 64,
         'tp_220': 88, 'tp_211': 160, 'tp_202': 232, 'tp_222': 304}
PW = 384


def _kron_eye(w, k):
    return jnp.kron(jnp.eye(k, dtype=w.dtype), w)


def _build_tc_params(params):
    wp = jnp.zeros((FW, PW), jnp.float32)
    wp = wp.at[0:16, 0:8].set(params['tp_000']['Wl'])
    wp = wp.at[0:16, 8:16].set(params['tp_011']['Wl'])
    wp = wp.at[16:40, 16:40].set(_kron_eye(params['tp_110']['Wl'], 3))
    wp = wp.at[16:40, 40:64].set(_kron_eye(params['tp_101']['Wl'], 3))
    wp = wp.at[16:40, 64:88].set(_kron_eye(params['tp_112']['Wl'], 3))
    wp = wp.at[40:76, 88:160].set(_kron_eye(params['tp_220']['Wl'], 9))
    wp = wp.at[40:76, 160:232].set(_kron_eye(params['tp_211']['Wl'], 9))
    wp = wp.at[40:76, 232:304].set(_kron_eye(params['tp_202']['Wl'], 9))
    wp = wp.at[40:76, 304:376].set(_kron_eye(params['tp_222']['Wl'], 9))
    wr = jnp.concatenate([params[t]['Wr'] for t in TP_ORDER], axis=1)
    woA = jnp.concatenate([params['tp_000']['Wo'], params['tp_110']['Wo'],
                           params['tp_220']['Wo']], axis=0)
    return dict(
        wpT=wp.T,
        wrT=wr.T,
        woA=woA.T,
        wo011=params['tp_011']['Wo'].T,
        wo101=params['tp_101']['Wo'].T,
        wo211=params['tp_211']['Wo'].T,
        wo112=params['tp_112']['Wo'].T,
        wo202=params['tp_202']['Wo'].T,
        wo222=params['tp_222']['Wo'].T,
    )


def _tc_body(g_ref, r_ref, wpT, wrT, woA, wo011, wo101, wo211, wo112,
             wo202, wo222, o_ref):
    f32 = jnp.float32
    dn_t = (((1,), (1,)), ((), ()))
    dn_m = (((1,), (0,)), ((), ()))
    dot = functools.partial(jax.lax.dot_general,
                            preferred_element_type=f32,
                            precision=jax.lax.Precision.HIGHEST)
    G = g_ref[...]
    Bsz = G.shape[0]
    r3 = r_ref[...][0:3, :]
    rsq = jnp.sum(r3 * r3, axis=0, keepdims=True) / R0
    srt = jnp.sqrt(rsq)
    npi = jnp.pi * jax.lax.broadcasted_iota(
        jnp.int32, (NRAD, 1), 0).astype(f32)
    env = jnp.maximum(1.0 - rsq, 0.0)
    rad = jnp.cos(npi * srt) * env
    y = r3 * (17.0 / R0)
    nrm = jnp.sqrt(jnp.sum(y * y, axis=0, keepdims=True) + 1e-12)
    rs = y * (jnp.tanh(nrm) / nrm)
    rs_l = [rs[i:i + 1, :] for i in range(3)]

    Gall = dot(wrT[...], rad, dn_m)
    P = dot(wpT[...], G, dn_t)
    g = {t: Gall[k * 8:(k + 1) * 8, :] for k, t in enumerate(TP_ORDER)}

    def pblk(t, k, n=8):
        o = _POFF[t] + k * n
        return P[o:o + n, :]

    h000 = pblk('tp_000', 0) * g['tp_000']
    d110 = sum(pblk('tp_110', i) * rs_l[i] for i in range(3))
    h110 = g['tp_110'] * d110
    d220 = sum(pblk('tp_220', i * 3 + j) * (rs_l[i] * rs_l[j])
               for i in range(3) for j in range(3))
    h220 = g['tp_220'] * d220
    psiA = dot(woA[...], jnp.concatenate([h000, h110, h220], axis=0), dn_m)

    t011 = dot(wo011[...], pblk('tp_011', 0) * g['tp_011'], dn_m)
    psiV_parts = []
    for i in range(3):
        v1 = dot(wo101[...], pblk('tp_101', i) * g['tp_101'], dn_m)
        w = sum(pblk('tp_211', i * 3 + j) * rs_l[j] for j in range(3))
        v2 = dot(wo211[...], g['tp_211'] * w, dn_m)
        psiV_parts.append(t011 * rs_l[i] + v1 + v2)
    psiV = jnp.concatenate(psiV_parts, axis=0)

    D = []
    for i in range(3):
        d1 = dot(wo112[...], pblk('tp_112', i) * g['tp_112'], dn_m)
        w2 = sum(pblk('tp_222', i * 3 + j) * rs_l[j] for j in range(3))
        d2 = dot(wo222[...], g['tp_222'] * w2, dn_m)
        D.append(d1 + d2)
    psiD_parts = []
    for i in range(3):
        for j in range(3):
            e202 = dot(wo202[...], pblk('tp_202', i * 3 + j) * g['tp_202'], dn_m)
            psiD_parts.append(e202 + D[i] * rs_l[j])
    psiD = jnp.concatenate(psiD_parts, axis=0)

    psi = jnp.concatenate(
        [psiA, psiV, psiD, jnp.zeros((4, Bsz), f32)], axis=0)
    o_ref[...] = dot(psi, jnp.eye(FW, dtype=f32),
                     (((0,), (0,)), ((), ())))


def _tc_compute(G, Rpad, tcp, block_e):
    E = G.shape[0]
    grid = (E // block_e,)
    full = lambda a: pl.BlockSpec(a.shape, lambda i: (0,) * a.ndim)
    porder = ('wpT', 'wrT', 'woA', 'wo011', 'wo101', 'wo211', 'wo112',
              'wo202', 'wo222')
    return pl.pallas_call(
        _tc_body,
        grid=grid,
        in_specs=[
            pl.BlockSpec((block_e, FW), lambda i: (i, 0)),
            pl.BlockSpec((8, block_e), lambda i: (0, i)),
        ] + [full(tcp[k]) for k in porder],
        out_specs=pl.BlockSpec((block_e, FW), lambda i: (i, 0)),
        out_shape=jax.ShapeDtypeStruct((E, FW), jnp.float32),
    )(G, Rpad, *[tcp[k] for k in porder])


def _sc_gather(table, idx):
    info = plsc.get_sparse_core_info()
    nw = info.num_cores * info.num_subcores
    E = idx.shape[0]
    per_w = E // nw
    C = 1000
    mesh = plsc.VectorSubcoreMesh(core_axis_name="c", subcore_axis_name="s")

    @functools.partial(
        pl.kernel, mesh=mesh,
        compiler_params=pltpu.CompilerParams(use_tc_tiling_on_sc=False),
        out_type=jax.ShapeDtypeStruct((E, FW), jnp.float32),
        scratch_types=[
            pltpu.VMEM((C,), jnp.int32),
            pltpu.VMEM((C, FW), jnp.float32),
            pltpu.SemaphoreType.DMA,
        ],
    )
    def k(table_hbm, idx_hbm, out_hbm, idx_v, rows_v, sem):
        wid = lax.axis_index("s") * info.num_cores + lax.axis_index("c")
        base = wid * per_w

        def body(j, carry):
            o = base + j * C
            pltpu.sync_copy(idx_hbm.at[pl.ds(o, C)], idx_v)
            pltpu.async_copy(table_hbm.at[idx_v], rows_v, sem).wait()
            pltpu.sync_copy(rows_v, out_hbm.at[pl.ds(o, C)])
            return carry

        lax.fori_loop(0, per_w // C, body, 0)

    return k(table, idx)


def _build_table(x_a, x_v, x_d):
    n = x_a.shape[0]
    xv = jnp.transpose(x_v, (0, 2, 1)).reshape(n, 24)
    xd = jnp.transpose(x_d, (0, 2, 3, 1)).reshape(n, 36)
    return jnp.concatenate(
        [x_a, xv, xd, jnp.zeros((n, 4), jnp.float32)], axis=1)


def _unpack_out(out):
    n = out.shape[0]
    B_a = out[:, 0:16]
    B_v = jnp.transpose(out[:, 16:40].reshape(n, 3, 8), (0, 2, 1))
    B_d = jnp.transpose(out[:, 40:76].reshape(n, 3, 3, 4), (0, 3, 1, 2))
    return B_a, B_v, B_d


def kernel(graph, r_ij, x_a, x_v, x_d, params):
    E = r_ij.shape[0]
    n = x_a.shape[0]
    src = graph[0]
    dst = graph[1]
    table = _build_table(x_a, x_v, x_d)
    Rpad = jnp.concatenate(
        [r_ij.T, jnp.zeros((5, E), jnp.float32)], axis=0)
    tcp = _build_tc_params(params)
    block_e = 1280 if E % 1280 == 0 else 16
    if E % 32000 == 0:
        G = _sc_gather(table, dst)
    else:
        G = jnp.take(table, dst, axis=0)
    psi = _tc_compute(G, Rpad, tcp, block_e)
    out = jax.ops.segment_sum(psi, src, num_segments=n)
    return _unpack_out(out)


# speedup vs baseline: 57.6551x; 1.0004x over previous
import functools

import jax
import jax.numpy as jnp
from jax import lax
from jax.experimental import pallas as pl
from jax.experimental.pallas import tpu as pltpu
from jax.experimental.pallas import tpu_sc as plsc

N_NODES = 50000
N_EDGES = 800000
DIM_A = 16
DIM_V = 8
DIM_D = 4
RANK = 8
R0 = 5.0
NRAD = 8
FW = 80

TP_ORDER = ('tp_000', 'tp_011', 'tp_110', 'tp_101', 'tp_112',
            'tp_220', 'tp_211', 'tp_202', 'tp_222')

_POFF = {'tp_000': 0, 'tp_011': 8, 'tp_110': 16, 'tp_101': 40, 'tp_112': 64,
         'tp_220': 88, 'tp_211': 160, 'tp_202': 232, 'tp_222': 304}
PW = 384


def _kron_eye(w, k):
    return jnp.kron(jnp.eye(k, dtype=w.dtype), w)


def _build_tc_params(params):
    wp = jnp.zeros((FW, PW), jnp.float32)
    wp = wp.at[0:16, 0:8].set(params['tp_000']['Wl'])
    wp = wp.at[0:16, 8:16].set(params['tp_011']['Wl'])
    wp = wp.at[16:40, 16:40].set(_kron_eye(params['tp_110']['Wl'], 3))
    wp = wp.at[16:40, 40:64].set(_kron_eye(params['tp_101']['Wl'], 3))
    wp = wp.at[16:40, 64:88].set(_kron_eye(params['tp_112']['Wl'], 3))
    wp = wp.at[40:76, 88:160].set(_kron_eye(params['tp_220']['Wl'], 9))
    wp = wp.at[40:76, 160:232].set(_kron_eye(params['tp_211']['Wl'], 9))
    wp = wp.at[40:76, 232:304].set(_kron_eye(params['tp_202']['Wl'], 9))
    wp = wp.at[40:76, 304:376].set(_kron_eye(params['tp_222']['Wl'], 9))
    wr = jnp.concatenate([params[t]['Wr'] for t in TP_ORDER], axis=1)
    woA = jnp.concatenate([params['tp_000']['Wo'], params['tp_110']['Wo'],
                           params['tp_220']['Wo']], axis=0)
    return dict(
        wpT=wp.T,
        wrT=wr.T,
        woA=woA.T,
        wo011=params['tp_011']['Wo'].T,
        wo101=params['tp_101']['Wo'].T,
        wo211=params['tp_211']['Wo'].T,
        wo112=params['tp_112']['Wo'].T,
        wo202=params['tp_202']['Wo'].T,
        wo222=params['tp_222']['Wo'].T,
    )


def _tc_body(g_ref, r_ref, wpT, wrT, woA, wo011, wo101, wo211, wo112,
             wo202, wo222, o_ref):
    f32 = jnp.float32
    dn_t = (((1,), (1,)), ((), ()))
    dn_m = (((1,), (0,)), ((), ()))
    dot = functools.partial(jax.lax.dot_general,
                            preferred_element_type=f32,
                            precision=jax.lax.Precision.HIGHEST)
    G = g_ref[...]
    Bsz = G.shape[0]
    r3 = r_ref[...][0:3, :]
    rsq = jnp.sum(r3 * r3, axis=0, keepdims=True) / R0
    srt = jnp.sqrt(rsq)
    npi = jnp.pi * jax.lax.broadcasted_iota(
        jnp.int32, (NRAD, 1), 0).astype(f32)
    env = jnp.maximum(1.0 - rsq, 0.0)
    rad = jnp.cos(npi * srt) * env
    y = r3 * (17.0 / R0)
    nrm = jnp.sqrt(jnp.sum(y * y, axis=0, keepdims=True) + 1e-12)
    rs = y * (jnp.tanh(nrm) / nrm)
    rs_l = [rs[i:i + 1, :] for i in range(3)]

    Gall = dot(wrT[...], rad, dn_m)
    P = dot(wpT[...], G, dn_t)
    g = {t: Gall[k * 8:(k + 1) * 8, :] for k, t in enumerate(TP_ORDER)}

    def pblk(t, k, n=8):
        o = _POFF[t] + k * n
        return P[o:o + n, :]

    h000 = pblk('tp_000', 0) * g['tp_000']
    d110 = sum(pblk('tp_110', i) * rs_l[i] for i in range(3))
    h110 = g['tp_110'] * d110
    d220 = sum(pblk('tp_220', i * 3 + j) * (rs_l[i] * rs_l[j])
               for i in range(3) for j in range(3))
    h220 = g['tp_220'] * d220
    psiA = dot(woA[...], jnp.concatenate([h000, h110, h220], axis=0), dn_m)

    t011 = dot(wo011[...], pblk('tp_011', 0) * g['tp_011'], dn_m)
    psiV_parts = []
    for i in range(3):
        v1 = dot(wo101[...], pblk('tp_101', i) * g['tp_101'], dn_m)
        w = sum(pblk('tp_211', i * 3 + j) * rs_l[j] for j in range(3))
        v2 = dot(wo211[...], g['tp_211'] * w, dn_m)
        psiV_parts.append(t011 * rs_l[i] + v1 + v2)
    psiV = jnp.concatenate(psiV_parts, axis=0)

    D = []
    for i in range(3):
        d1 = dot(wo112[...], pblk('tp_112', i) * g['tp_112'], dn_m)
        w2 = sum(pblk('tp_222', i * 3 + j) * rs_l[j] for j in range(3))
        d2 = dot(wo222[...], g['tp_222'] * w2, dn_m)
        D.append(d1 + d2)
    psiD_parts = []
    for i in range(3):
        for j in range(3):
            e202 = dot(wo202[...], pblk('tp_202', i * 3 + j) * g['tp_202'], dn_m)
            psiD_parts.append(e202 + D[i] * rs_l[j])
    psiD = jnp.concatenate(psiD_parts, axis=0)

    psi = jnp.concatenate(
        [psiA, psiV, psiD, jnp.zeros((4, Bsz), f32)], axis=0)
    o_ref[...] = dot(psi, jnp.eye(FW, dtype=f32),
                     (((0,), (0,)), ((), ())))


def _tc_compute(G, Rpad, tcp, block_e):
    E = G.shape[0]
    grid = (E // block_e,)
    full = lambda a: pl.BlockSpec(a.shape, lambda i: (0,) * a.ndim)
    porder = ('wpT', 'wrT', 'woA', 'wo011', 'wo101', 'wo211', 'wo112',
              'wo202', 'wo222')
    return pl.pallas_call(
        _tc_body,
        grid=grid,
        in_specs=[
            pl.BlockSpec((block_e, FW), lambda i: (i, 0)),
            pl.BlockSpec((8, block_e), lambda i: (0, i)),
        ] + [full(tcp[k]) for k in porder],
        out_specs=pl.BlockSpec((block_e, FW), lambda i: (i, 0)),
        out_shape=jax.ShapeDtypeStruct((E, FW), jnp.float32),
    )(G, Rpad, *[tcp[k] for k in porder])


def _sc_gather(table, idx):
    info = plsc.get_sparse_core_info()
    nw = info.num_cores * info.num_subcores
    E = idx.shape[0]
    per_w = E // nw
    C = 1000
    mesh = plsc.VectorSubcoreMesh(core_axis_name="c", subcore_axis_name="s")

    @functools.partial(
        pl.kernel, mesh=mesh,
        compiler_params=pltpu.CompilerParams(use_tc_tiling_on_sc=False),
        out_type=jax.ShapeDtypeStruct((E, FW), jnp.float32),
        scratch_types=[
            pltpu.VMEM((C,), jnp.int32),
            pltpu.VMEM((C, FW), jnp.float32),
            pltpu.SemaphoreType.DMA,
        ],
    )
    def k(table_hbm, idx_hbm, out_hbm, idx_v, rows_v, sem):
        wid = lax.axis_index("s") * info.num_cores + lax.axis_index("c")
        base = wid * per_w

        def body(j, carry):
            o = base + j * C
            pltpu.sync_copy(idx_hbm.at[pl.ds(o, C)], idx_v)
            pltpu.async_copy(table_hbm.at[idx_v], rows_v, sem).wait()
            pltpu.sync_copy(rows_v, out_hbm.at[pl.ds(o, C)])
            return carry

        lax.fori_loop(0, per_w // C, body, 0)

    return k(table, idx)


def _sc_scatter_add(psi, src, n):
    info = plsc.get_sparse_core_info()
    ns = info.num_subcores
    E = psi.shape[0]
    per_w = E // ns
    C = 1000
    CH = per_w // C
    NZ = n // C
    half = FW // 2
    mesh = plsc.VectorSubcoreMesh(core_axis_name="c", subcore_axis_name="s")

    @functools.partial(
        pl.kernel, mesh=mesh,
        compiler_params=pltpu.CompilerParams(use_tc_tiling_on_sc=False),
        out_type=jax.ShapeDtypeStruct((n, FW), jnp.float32),
        scratch_types=[
            pltpu.VMEM((C,), jnp.int32),
            pltpu.VMEM((C, half), jnp.float32),
        ],
    )
    def k(psi_hbm, idx_hbm, zero_hbm, out_hbm, idx_v, buf):
        c = lax.axis_index("c")
        s = lax.axis_index("s")
        nz_own = (NZ + ns - 1) // ns

        def run(acc):
            def zbody(z, carry):
                zi = z * ns + s

                @pl.when(zi < NZ)
                def _():
                    pltpu.sync_copy(zero_hbm, acc.at[pl.ds(zi * C, C), :])
                return carry
            lax.fori_loop(0, nz_own, zbody, 0)
            plsc.subcore_barrier()

            base = s * per_w

            def body(j, carry):
                o = base + j * C
                pltpu.sync_copy(idx_hbm.at[pl.ds(o, C)], idx_v)
                pltpu.sync_copy(
                    psi_hbm.at[pl.ds(o, C), pl.ds(c * half, half)], buf)
                pltpu.sync_copy(buf, acc.at[idx_v], add=True)
                return carry
            lax.fori_loop(0, CH, body, 0)
            plsc.subcore_barrier()

            def wbody(z, carry):
                zi = z * ns + s

                @pl.when(zi < NZ)
                def _():
                    pltpu.sync_copy(
                        acc.at[pl.ds(zi * C, C), :],
                        out_hbm.at[pl.ds(zi * C, C), pl.ds(c * half, half)])
                return carry
            lax.fori_loop(0, nz_own, wbody, 0)

        pl.run_scoped(run, plsc.MemoryRef((n, half), jnp.float32,
                                          pltpu.VMEM_SHARED))

    zero = jnp.zeros((C, half), jnp.float32)
    return k(psi, src, zero)


def _build_table(x_a, x_v, x_d):
    n = x_a.shape[0]
    xv = jnp.transpose(x_v, (0, 2, 1)).reshape(n, 24)
    xd = jnp.transpose(x_d, (0, 2, 3, 1)).reshape(n, 36)
    return jnp.concatenate(
        [x_a, xv, xd, jnp.zeros((n, 4), jnp.float32)], axis=1)


def _unpack_out(out):
    n = out.shape[0]
    B_a = out[:, 0:16]
    B_v = jnp.transpose(out[:, 16:40].reshape(n, 3, 8), (0, 2, 1))
    B_d = jnp.transpose(out[:, 40:76].reshape(n, 3, 3, 4), (0, 3, 1, 2))
    return B_a, B_v, B_d


def kernel(graph, r_ij, x_a, x_v, x_d, params):
    E = r_ij.shape[0]
    n = x_a.shape[0]
    src = graph[0]
    dst = graph[1]
    table = _build_table(x_a, x_v, x_d)
    Rpad = jnp.concatenate(
        [r_ij.T, jnp.zeros((5, E), jnp.float32)], axis=0)
    tcp = _build_tc_params(params)
    block_e = 1280 if E % 1280 == 0 else 16
    if E % 32000 == 0:
        G = _sc_gather(table, dst)
    else:
        G = jnp.take(table, dst, axis=0)
    psi = _tc_compute(G, Rpad, tcp, block_e)
    if E % 32000 == 0 and n % 16000 == 0:
        out = _sc_scatter_add(psi, src, n)
    else:
        out = jax.ops.segment_sum(psi, src, num_segments=n)
    return _unpack_out(out)


# speedup vs baseline: 72.9012x; 1.2644x over previous
import functools

import jax
import jax.numpy as jnp
from jax import lax
from jax.experimental import pallas as pl
from jax.experimental.pallas import tpu as pltpu
from jax.experimental.pallas import tpu_sc as plsc

N_NODES = 50000
N_EDGES = 800000
DIM_A = 16
DIM_V = 8
DIM_D = 4
RANK = 8
R0 = 5.0
NRAD = 8
FW = 80

TP_ORDER = ('tp_000', 'tp_011', 'tp_110', 'tp_101', 'tp_112',
            'tp_220', 'tp_211', 'tp_202', 'tp_222')

_POFF = {'tp_000': 0, 'tp_011': 8, 'tp_110': 16, 'tp_101': 40, 'tp_112': 64,
         'tp_220': 88, 'tp_211': 160, 'tp_202': 232, 'tp_222': 304}
PW = 384


def _kron_eye(w, k):
    return jnp.kron(jnp.eye(k, dtype=w.dtype), w)


def _build_tc_params(params):
    wp = jnp.zeros((FW, PW), jnp.float32)
    wp = wp.at[0:16, 0:8].set(params['tp_000']['Wl'])
    wp = wp.at[0:16, 8:16].set(params['tp_011']['Wl'])
    wp = wp.at[16:40, 16:40].set(_kron_eye(params['tp_110']['Wl'], 3))
    wp = wp.at[16:40, 40:64].set(_kron_eye(params['tp_101']['Wl'], 3))
    wp = wp.at[16:40, 64:88].set(_kron_eye(params['tp_112']['Wl'], 3))
    wp = wp.at[40:76, 88:160].set(_kron_eye(params['tp_220']['Wl'], 9))
    wp = wp.at[40:76, 160:232].set(_kron_eye(params['tp_211']['Wl'], 9))
    wp = wp.at[40:76, 232:304].set(_kron_eye(params['tp_202']['Wl'], 9))
    wp = wp.at[40:76, 304:376].set(_kron_eye(params['tp_222']['Wl'], 9))
    wr = jnp.concatenate([params[t]['Wr'] for t in TP_ORDER], axis=1)
    woA = jnp.concatenate([params['tp_000']['Wo'], params['tp_110']['Wo'],
                           params['tp_220']['Wo']], axis=0)
    return dict(
        wpT=wp.T,
        wrT=wr.T,
        woA=woA.T,
        wo011=params['tp_011']['Wo'].T,
        wo101=params['tp_101']['Wo'].T,
        wo211=params['tp_211']['Wo'].T,
        wo112=params['tp_112']['Wo'].T,
        wo202=params['tp_202']['Wo'].T,
        wo222=params['tp_222']['Wo'].T,
    )


def _tc_body(g_ref, r_ref, wpT, wrT, woA, wo011, wo101, wo211, wo112,
             wo202, wo222, o_ref):
    f32 = jnp.float32
    dn_t = (((1,), (1,)), ((), ()))
    dn_m = (((1,), (0,)), ((), ()))
    dot = functools.partial(jax.lax.dot_general,
                            preferred_element_type=f32,
                            precision=jax.lax.Precision.HIGHEST)
    G = g_ref[...]
    Bsz = G.shape[0]
    r3 = r_ref[...][0:3, :]
    rsq = jnp.sum(r3 * r3, axis=0, keepdims=True) / R0
    srt = jnp.sqrt(rsq)
    npi = jnp.pi * jax.lax.broadcasted_iota(
        jnp.int32, (NRAD, 1), 0).astype(f32)
    env = jnp.maximum(1.0 - rsq, 0.0)
    rad = jnp.cos(npi * srt) * env
    y = r3 * (17.0 / R0)
    nrm = jnp.sqrt(jnp.sum(y * y, axis=0, keepdims=True) + 1e-12)
    rs = y * (jnp.tanh(nrm) / nrm)
    rs_l = [rs[i:i + 1, :] for i in range(3)]

    Gall = dot(wrT[...], rad, dn_m)
    P = dot(wpT[...], G, dn_t)
    g = {t: Gall[k * 8:(k + 1) * 8, :] for k, t in enumerate(TP_ORDER)}

    def pblk(t, k, n=8):
        o = _POFF[t] + k * n
        return P[o:o + n, :]

    h000 = pblk('tp_000', 0) * g['tp_000']
    d110 = sum(pblk('tp_110', i) * rs_l[i] for i in range(3))
    h110 = g['tp_110'] * d110
    d220 = sum(pblk('tp_220', i * 3 + j) * (rs_l[i] * rs_l[j])
               for i in range(3) for j in range(3))
    h220 = g['tp_220'] * d220
    psiA = dot(woA[...], jnp.concatenate([h000, h110, h220], axis=0), dn_m)

    t011 = dot(wo011[...], pblk('tp_011', 0) * g['tp_011'], dn_m)
    psiV_parts = []
    for i in range(3):
        v1 = dot(wo101[...], pblk('tp_101', i) * g['tp_101'], dn_m)
        w = sum(pblk('tp_211', i * 3 + j) * rs_l[j] for j in range(3))
        v2 = dot(wo211[...], g['tp_211'] * w, dn_m)
        psiV_parts.append(t011 * rs_l[i] + v1 + v2)
    psiV = jnp.concatenate(psiV_parts, axis=0)

    D = []
    for i in range(3):
        d1 = dot(wo112[...], pblk('tp_112', i) * g['tp_112'], dn_m)
        w2 = sum(pblk('tp_222', i * 3 + j) * rs_l[j] for j in range(3))
        d2 = dot(wo222[...], g['tp_222'] * w2, dn_m)
        D.append(d1 + d2)
    psiD_parts = []
    for i in range(3):
        for j in range(3):
            e202 = dot(wo202[...], pblk('tp_202', i * 3 + j) * g['tp_202'], dn_m)
            psiD_parts.append(e202 + D[i] * rs_l[j])
    psiD = jnp.concatenate(psiD_parts, axis=0)

    psi = jnp.concatenate(
        [psiA, psiV, psiD, jnp.zeros((4, Bsz), f32)], axis=0)
    o_ref[...] = dot(psi, jnp.eye(FW, dtype=f32),
                     (((0,), (0,)), ((), ())))


def _tc_compute(G, Rpad, tcp, block_e):
    E = G.shape[0]
    grid = (E // block_e,)
    full = lambda a: pl.BlockSpec(a.shape, lambda i: (0,) * a.ndim)
    porder = ('wpT', 'wrT', 'woA', 'wo011', 'wo101', 'wo211', 'wo112',
              'wo202', 'wo222')
    return pl.pallas_call(
        _tc_body,
        grid=grid,
        in_specs=[
            pl.BlockSpec((block_e, FW), lambda i: (i, 0)),
            pl.BlockSpec((8, block_e), lambda i: (0, i)),
        ] + [full(tcp[k]) for k in porder],
        out_specs=pl.BlockSpec((block_e, FW), lambda i: (i, 0)),
        out_shape=jax.ShapeDtypeStruct((E, FW), jnp.float32),
    )(G, Rpad, *[tcp[k] for k in porder])


def _sc_gather(table, idx):
    info = plsc.get_sparse_core_info()
    nw = info.num_cores * info.num_subcores
    E = idx.shape[0]
    per_w = E // nw
    C = 1000
    mesh = plsc.VectorSubcoreMesh(core_axis_name="c", subcore_axis_name="s")

    @functools.partial(
        pl.kernel, mesh=mesh,
        compiler_params=pltpu.CompilerParams(use_tc_tiling_on_sc=False),
        out_type=jax.ShapeDtypeStruct((E, FW), jnp.float32),
        scratch_types=[
            pltpu.VMEM((C,), jnp.int32),
            pltpu.VMEM((C, FW), jnp.float32),
            pltpu.SemaphoreType.DMA,
        ],
    )
    def k(table_hbm, idx_hbm, out_hbm, idx_v, rows_v, sem):
        wid = lax.axis_index("s") * info.num_cores + lax.axis_index("c")
        base = wid * per_w

        def body(j, carry):
            o = base + j * C
            pltpu.sync_copy(idx_hbm.at[pl.ds(o, C)], idx_v)
            pltpu.async_copy(table_hbm.at[idx_v], rows_v, sem).wait()
            pltpu.sync_copy(rows_v, out_hbm.at[pl.ds(o, C)])
            return carry

        lax.fori_loop(0, per_w // C, body, 0)

    return k(table, idx)


def _sc_scatter_add(psi, src, n):
    info = plsc.get_sparse_core_info()
    ns = info.num_subcores
    E = psi.shape[0]
    per_w = E // ns
    C = 128
    NCH = E // C
    CZ = 1000
    NZ = n // CZ
    half = FW // 2
    mesh = plsc.VectorSubcoreMesh(core_axis_name="c", subcore_axis_name="s")

    @functools.partial(
        pl.kernel, mesh=mesh,
        compiler_params=pltpu.CompilerParams(use_tc_tiling_on_sc=False),
        out_type=jax.ShapeDtypeStruct((n, FW), jnp.float32),
        scratch_types=[
            pltpu.VMEM((C,), jnp.int32),
            pltpu.VMEM((C, half), jnp.float32),
            pltpu.VMEM_SHARED((n, half), jnp.float32),
        ],
    )
    def k(psi_hbm, idx_hbm, zero_hbm, out_hbm, idx_v, buf, acc):
        c = lax.axis_index("c")
        s = lax.axis_index("s")
        nz_own = (NZ + ns - 1) // ns

        def run(acc):
            def zbody(z, carry):
                zi = z * ns + s

                @pl.when(zi < NZ)
                def _():
                    pltpu.sync_copy(zero_hbm, acc.at[pl.ds(zi * CZ, CZ), :])
                return carry
            lax.fori_loop(0, nz_own, zbody, 0)
            plsc.subcore_barrier()

            def body(j, carry):
                k = j * ns + s

                @pl.when(k < NCH)
                def _():
                    o = k * C
                    pltpu.sync_copy(idx_hbm.at[pl.ds(o, C)], idx_v)
                    pltpu.sync_copy(
                        psi_hbm.at[pl.ds(o, C), pl.ds(c * half, half)], buf)
                    pltpu.sync_copy(buf, acc.at[idx_v], add=True)
                return carry
            lax.fori_loop(0, (NCH + ns - 1) // ns, body, 0)
            plsc.subcore_barrier()

            def wbody(z, carry):
                zi = z * ns + s

                @pl.when(zi < NZ)
                def _():
                    pltpu.sync_copy(
                        acc.at[pl.ds(zi * CZ, CZ), :],
                        out_hbm.at[pl.ds(zi * CZ, CZ), pl.ds(c * half, half)])
                return carry
            lax.fori_loop(0, nz_own, wbody, 0)

        run(acc)

    zero = jnp.zeros((CZ, half), jnp.float32)
    return k(psi, src, zero)


def _build_table(x_a, x_v, x_d):
    n = x_a.shape[0]
    xv = jnp.transpose(x_v, (0, 2, 1)).reshape(n, 24)
    xd = jnp.transpose(x_d, (0, 2, 3, 1)).reshape(n, 36)
    return jnp.concatenate(
        [x_a, xv, xd, jnp.zeros((n, 4), jnp.float32)], axis=1)


def _unpack_out(out):
    n = out.shape[0]
    B_a = out[:, 0:16]
    B_v = jnp.transpose(out[:, 16:40].reshape(n, 3, 8), (0, 2, 1))
    B_d = jnp.transpose(out[:, 40:76].reshape(n, 3, 3, 4), (0, 3, 1, 2))
    return B_a, B_v, B_d


def kernel(graph, r_ij, x_a, x_v, x_d, params):
    E = r_ij.shape[0]
    n = x_a.shape[0]
    src = graph[0]
    dst = graph[1]
    table = _build_table(x_a, x_v, x_d)
    Rpad = jnp.concatenate(
        [r_ij.T, jnp.zeros((5, E), jnp.float32)], axis=0)
    tcp = _build_tc_params(params)
    block_e = 1280 if E % 1280 == 0 else 16
    if E % 32000 == 0:
        G = _sc_gather(table, dst)
    else:
        G = jnp.take(table, dst, axis=0)
    psi = _tc_compute(G, Rpad, tcp, block_e)
    if E % 32000 == 0 and n % 1000 == 0:
        out = _sc_scatter_add(psi, src, n)
    else:
        out = jax.ops.segment_sum(psi, src, num_segments=n)
    return _unpack_out(out)


# speedup vs baseline: 135.9740x; 1.8652x over previous
import functools

import jax
import jax.numpy as jnp
from jax import lax
from jax.experimental import pallas as pl
from jax.experimental.pallas import tpu as pltpu
from jax.experimental.pallas import tpu_sc as plsc

N_NODES = 50000
N_EDGES = 800000
DIM_A = 16
DIM_V = 8
DIM_D = 4
RANK = 8
R0 = 5.0
NRAD = 8
FW = 80

TP_ORDER = ('tp_000', 'tp_011', 'tp_110', 'tp_101', 'tp_112',
            'tp_220', 'tp_211', 'tp_202', 'tp_222')

_POFF = {'tp_000': 0, 'tp_011': 8, 'tp_110': 16, 'tp_101': 40, 'tp_112': 64,
         'tp_220': 88, 'tp_211': 160, 'tp_202': 232, 'tp_222': 304}
PW = 384


def _kron_eye(w, k):
    return jnp.kron(jnp.eye(k, dtype=w.dtype), w)


def _build_tc_params(params):
    wp = jnp.zeros((FW, PW), jnp.float32)
    wp = wp.at[0:16, 0:8].set(params['tp_000']['Wl'])
    wp = wp.at[0:16, 8:16].set(params['tp_011']['Wl'])
    wp = wp.at[16:40, 16:40].set(_kron_eye(params['tp_110']['Wl'], 3))
    wp = wp.at[16:40, 40:64].set(_kron_eye(params['tp_101']['Wl'], 3))
    wp = wp.at[16:40, 64:88].set(_kron_eye(params['tp_112']['Wl'], 3))
    wp = wp.at[40:76, 88:160].set(_kron_eye(params['tp_220']['Wl'], 9))
    wp = wp.at[40:76, 160:232].set(_kron_eye(params['tp_211']['Wl'], 9))
    wp = wp.at[40:76, 232:304].set(_kron_eye(params['tp_202']['Wl'], 9))
    wp = wp.at[40:76, 304:376].set(_kron_eye(params['tp_222']['Wl'], 9))
    wr = jnp.concatenate([params[t]['Wr'] for t in TP_ORDER], axis=1)
    woA = jnp.concatenate([params['tp_000']['Wo'], params['tp_110']['Wo'],
                           params['tp_220']['Wo']], axis=0)
    return dict(
        wpT=wp.T,
        wrT=wr.T,
        woA=woA.T,
        wo011=params['tp_011']['Wo'].T,
        wo101=params['tp_101']['Wo'].T,
        wo211=params['tp_211']['Wo'].T,
        wo112=params['tp_112']['Wo'].T,
        wo202=params['tp_202']['Wo'].T,
        wo222=params['tp_222']['Wo'].T,
    )


def _tc_body(g_ref, r_ref, wpT, wrT, woA, wo011, wo101, wo211, wo112,
             wo202, wo222, o_ref):
    f32 = jnp.float32
    dn_t = (((1,), (1,)), ((), ()))
    dn_m = (((1,), (0,)), ((), ()))
    dot = functools.partial(jax.lax.dot_general,
                            preferred_element_type=f32,
                            precision=jax.lax.Precision.DEFAULT)
    G = g_ref[...]
    Bsz = G.shape[0]
    r3 = r_ref[...][0:3, :]
    rsq = jnp.sum(r3 * r3, axis=0, keepdims=True) / R0
    srt = jnp.sqrt(rsq)
    npi = jnp.pi * jax.lax.broadcasted_iota(
        jnp.int32, (NRAD, 1), 0).astype(f32)
    env = jnp.maximum(1.0 - rsq, 0.0)
    rad = jnp.cos(npi * srt) * env
    y = r3 * (17.0 / R0)
    nrm = jnp.sqrt(jnp.sum(y * y, axis=0, keepdims=True) + 1e-12)
    rs = y * (jnp.tanh(nrm) / nrm)
    rs_l = [rs[i:i + 1, :] for i in range(3)]

    Gall = dot(wrT[...], rad, dn_m)
    P = dot(wpT[...], G, dn_t)
    g = {t: Gall[k * 8:(k + 1) * 8, :] for k, t in enumerate(TP_ORDER)}

    def pblk(t, k, n=8):
        o = _POFF[t] + k * n
        return P[o:o + n, :]

    h000 = pblk('tp_000', 0) * g['tp_000']
    d110 = sum(pblk('tp_110', i) * rs_l[i] for i in range(3))
    h110 = g['tp_110'] * d110
    d220 = sum(pblk('tp_220', i * 3 + j) * (rs_l[i] * rs_l[j])
               for i in range(3) for j in range(3))
    h220 = g['tp_220'] * d220
    psiA = dot(woA[...], jnp.concatenate([h000, h110, h220], axis=0), dn_m)

    t011 = dot(wo011[...], pblk('tp_011', 0) * g['tp_011'], dn_m)
    psiV_parts = []
    for i in range(3):
        v1 = dot(wo101[...], pblk('tp_101', i) * g['tp_101'], dn_m)
        w = sum(pblk('tp_211', i * 3 + j) * rs_l[j] for j in range(3))
        v2 = dot(wo211[...], g['tp_211'] * w, dn_m)
        psiV_parts.append(t011 * rs_l[i] + v1 + v2)
    psiV = jnp.concatenate(psiV_parts, axis=0)

    D = []
    for i in range(3):
        d1 = dot(wo112[...], pblk('tp_112', i) * g['tp_112'], dn_m)
        w2 = sum(pblk('tp_222', i * 3 + j) * rs_l[j] for j in range(3))
        d2 = dot(wo222[...], g['tp_222'] * w2, dn_m)
        D.append(d1 + d2)
    psiD_parts = []
    for i in range(3):
        for j in range(3):
            e202 = dot(wo202[...], pblk('tp_202', i * 3 + j) * g['tp_202'], dn_m)
            psiD_parts.append(e202 + D[i] * rs_l[j])
    psiD = jnp.concatenate(psiD_parts, axis=0)

    psi = jnp.concatenate(
        [psiA, psiV, psiD, jnp.zeros((4, Bsz), f32)], axis=0)
    o_ref[...] = dot(psi, jnp.eye(FW, dtype=f32),
                     (((0,), (0,)), ((), ())))


def _tc_compute(G, Rpad, tcp, block_e):
    E = G.shape[0]
    grid = (E // block_e,)
    full = lambda a: pl.BlockSpec(a.shape, lambda i: (0,) * a.ndim)
    porder = ('wpT', 'wrT', 'woA', 'wo011', 'wo101', 'wo211', 'wo112',
              'wo202', 'wo222')
    return pl.pallas_call(
        _tc_body,
        grid=grid,
        in_specs=[
            pl.BlockSpec((block_e, FW), lambda i: (i, 0)),
            pl.BlockSpec((8, block_e), lambda i: (0, i)),
        ] + [full(tcp[k]) for k in porder],
        out_specs=pl.BlockSpec((block_e, FW), lambda i: (i, 0)),
        out_shape=jax.ShapeDtypeStruct((E, FW), jnp.float32),
    )(G, Rpad, *[tcp[k] for k in porder])


def _sc_gather(table, idx):
    info = plsc.get_sparse_core_info()
    nw = info.num_cores * info.num_subcores
    E = idx.shape[0]
    per_w = E // nw
    C = 1000
    mesh = plsc.VectorSubcoreMesh(core_axis_name="c", subcore_axis_name="s")

    @functools.partial(
        pl.kernel, mesh=mesh,
        compiler_params=pltpu.CompilerParams(use_tc_tiling_on_sc=False),
        out_type=jax.ShapeDtypeStruct((E, FW), jnp.float32),
        scratch_types=[
            pltpu.VMEM((C,), jnp.int32),
            pltpu.VMEM((C, FW), jnp.float32),
            pltpu.SemaphoreType.DMA,
        ],
    )
    def k(table_hbm, idx_hbm, out_hbm, idx_v, rows_v, sem):
        wid = lax.axis_index("s") * info.num_cores + lax.axis_index("c")
        base = wid * per_w

        def body(j, carry):
            o = base + j * C
            pltpu.sync_copy(idx_hbm.at[pl.ds(o, C)], idx_v)
            pltpu.async_copy(table_hbm.at[idx_v], rows_v, sem).wait()
            pltpu.sync_copy(rows_v, out_hbm.at[pl.ds(o, C)])
            return carry

        lax.fori_loop(0, per_w // C, body, 0)

    return k(table, idx)


def _sc_scatter_add(psi, src, n):
    info = plsc.get_sparse_core_info()
    ns = info.num_subcores
    E = psi.shape[0]
    per_w = E // ns
    C = 128
    NCH = E // C
    CZ = 1000
    NZ = n // CZ
    half = FW // 2
    mesh = plsc.VectorSubcoreMesh(core_axis_name="c", subcore_axis_name="s")

    @functools.partial(
        pl.kernel, mesh=mesh,
        compiler_params=pltpu.CompilerParams(use_tc_tiling_on_sc=False),
        out_type=jax.ShapeDtypeStruct((n, FW), jnp.float32),
        scratch_types=[
            pltpu.VMEM((C,), jnp.int32),
            pltpu.VMEM((C, half), jnp.float32),
            pltpu.VMEM_SHARED((n, half), jnp.float32),
        ],
    )
    def k(psi_hbm, idx_hbm, zero_hbm, out_hbm, idx_v, buf, acc):
        c = lax.axis_index("c")
        s = lax.axis_index("s")
        nz_own = (NZ + ns - 1) // ns

        def run(acc):
            def zbody(z, carry):
                zi = z * ns + s

                @pl.when(zi < NZ)
                def _():
                    pltpu.sync_copy(zero_hbm, acc.at[pl.ds(zi * CZ, CZ), :])
                return carry
            lax.fori_loop(0, nz_own, zbody, 0)
            plsc.subcore_barrier()

            def body(j, carry):
                k = j * ns + s

                @pl.when(k < NCH)
                def _():
                    o = k * C
                    pltpu.sync_copy(idx_hbm.at[pl.ds(o, C)], idx_v)
                    pltpu.sync_copy(
                        psi_hbm.at[pl.ds(o, C), pl.ds(c * half, half)], buf)
                    pltpu.sync_copy(buf, acc.at[idx_v], add=True)
                return carry
            lax.fori_loop(0, (NCH + ns - 1) // ns, body, 0)
            plsc.subcore_barrier()

            def wbody(z, carry):
                zi = z * ns + s

                @pl.when(zi < NZ)
                def _():
                    pltpu.sync_copy(
                        acc.at[pl.ds(zi * CZ, CZ), :],
                        out_hbm.at[pl.ds(zi * CZ, CZ), pl.ds(c * half, half)])
                return carry
            lax.fori_loop(0, nz_own, wbody, 0)

        run(acc)

    zero = jnp.zeros((CZ, half), jnp.float32)
    return k(psi, src, zero)


def _build_table(x_a, x_v, x_d):
    n = x_a.shape[0]
    xv = jnp.transpose(x_v, (0, 2, 1)).reshape(n, 24)
    xd = jnp.transpose(x_d, (0, 2, 3, 1)).reshape(n, 36)
    return jnp.concatenate(
        [x_a, xv, xd, jnp.zeros((n, 4), jnp.float32)], axis=1)


def _unpack_out(out):
    n = out.shape[0]
    B_a = out[:, 0:16]
    B_v = jnp.transpose(out[:, 16:40].reshape(n, 3, 8), (0, 2, 1))
    B_d = jnp.transpose(out[:, 40:76].reshape(n, 3, 3, 4), (0, 3, 1, 2))
    return B_a, B_v, B_d


def kernel(graph, r_ij, x_a, x_v, x_d, params):
    E = r_ij.shape[0]
    n = x_a.shape[0]
    src = graph[0]
    dst = graph[1]
    table = _build_table(x_a, x_v, x_d)
    Rpad = jnp.concatenate(
        [r_ij.T, jnp.zeros((5, E), jnp.float32)], axis=0)
    tcp = _build_tc_params(params)
    block_e = 1280 if E % 1280 == 0 else 16
    if E % 32000 == 0:
        G = _sc_gather(table, dst)
    else:
        G = jnp.take(table, dst, axis=0)
    psi = _tc_compute(G, Rpad, tcp, block_e)
    if E % 32000 == 0 and n % 1000 == 0:
        out = _sc_scatter_add(psi, src, n)
    else:
        out = jax.ops.segment_sum(psi, src, num_segments=n)
    return _unpack_out(out)


# speedup vs baseline: 151.1968x; 1.1120x over previous
import functools

import jax
import jax.numpy as jnp
from jax import lax
from jax.experimental import pallas as pl
from jax.experimental.pallas import tpu as pltpu
from jax.experimental.pallas import tpu_sc as plsc

N_NODES = 50000
N_EDGES = 800000
DIM_A = 16
DIM_V = 8
DIM_D = 4
RANK = 8
R0 = 5.0
NRAD = 8
FW = 80

TP_ORDER = ('tp_000', 'tp_011', 'tp_110', 'tp_101', 'tp_112',
            'tp_220', 'tp_211', 'tp_202', 'tp_222')

_POFF = {'tp_000': 0, 'tp_011': 8, 'tp_110': 16, 'tp_101': 40, 'tp_112': 64,
         'tp_220': 88, 'tp_211': 160, 'tp_202': 232, 'tp_222': 304}
PW = 384


def _kron_eye(w, k):
    return jnp.kron(jnp.eye(k, dtype=w.dtype), w)


def _build_tc_params(params):
    wp = jnp.zeros((FW, PW), jnp.float32)
    wp = wp.at[0:16, 0:8].set(params['tp_000']['Wl'])
    wp = wp.at[0:16, 8:16].set(params['tp_011']['Wl'])
    wp = wp.at[16:40, 16:40].set(_kron_eye(params['tp_110']['Wl'], 3))
    wp = wp.at[16:40, 40:64].set(_kron_eye(params['tp_101']['Wl'], 3))
    wp = wp.at[16:40, 64:88].set(_kron_eye(params['tp_112']['Wl'], 3))
    wp = wp.at[40:76, 88:160].set(_kron_eye(params['tp_220']['Wl'], 9))
    wp = wp.at[40:76, 160:232].set(_kron_eye(params['tp_211']['Wl'], 9))
    wp = wp.at[40:76, 232:304].set(_kron_eye(params['tp_202']['Wl'], 9))
    wp = wp.at[40:76, 304:376].set(_kron_eye(params['tp_222']['Wl'], 9))
    wr = jnp.concatenate([params[t]['Wr'] for t in TP_ORDER], axis=1)
    woA = jnp.concatenate([params['tp_000']['Wo'], params['tp_110']['Wo'],
                           params['tp_220']['Wo']], axis=0)
    return dict(
        wpT=wp.T,
        wrT=wr.T,
        woA=woA.T,
        wo011=params['tp_011']['Wo'].T,
        wo101=params['tp_101']['Wo'].T,
        wo211=params['tp_211']['Wo'].T,
        wo112=params['tp_112']['Wo'].T,
        wo202=params['tp_202']['Wo'].T,
        wo222=params['tp_222']['Wo'].T,
    )


def _tc_body(g_ref, r_ref, wpT, wrT, woA, wo011, wo101, wo211, wo112,
             wo202, wo222, o_ref):
    f32 = jnp.float32
    dn_t = (((1,), (1,)), ((), ()))
    dn_m = (((1,), (0,)), ((), ()))
    dot = functools.partial(jax.lax.dot_general,
                            preferred_element_type=f32,
                            precision=jax.lax.Precision.DEFAULT)
    G = g_ref[...]
    Bsz = G.shape[0]
    r3 = r_ref[...][0:3, :]
    rsq = jnp.sum(r3 * r3, axis=0, keepdims=True) / R0
    srt = jnp.sqrt(rsq)
    npi = jnp.pi * jax.lax.broadcasted_iota(
        jnp.int32, (NRAD, 1), 0).astype(f32)
    env = jnp.maximum(1.0 - rsq, 0.0)
    rad = jnp.cos(npi * srt) * env
    y = r3 * (17.0 / R0)
    nrm = jnp.sqrt(jnp.sum(y * y, axis=0, keepdims=True) + 1e-12)
    rs = y * (jnp.tanh(nrm) / nrm)
    rs_l = [rs[i:i + 1, :] for i in range(3)]

    Gall = dot(wrT[...], rad, dn_m)
    P = dot(wpT[...], G, dn_t)
    g = {t: Gall[k * 8:(k + 1) * 8, :] for k, t in enumerate(TP_ORDER)}

    def pblk(t, k, n=8):
        o = _POFF[t] + k * n
        return P[o:o + n, :]

    h000 = pblk('tp_000', 0) * g['tp_000']
    d110 = sum(pblk('tp_110', i) * rs_l[i] for i in range(3))
    h110 = g['tp_110'] * d110
    d220 = sum(pblk('tp_220', i * 3 + j) * (rs_l[i] * rs_l[j])
               for i in range(3) for j in range(3))
    h220 = g['tp_220'] * d220
    psiA = dot(woA[...], jnp.concatenate([h000, h110, h220], axis=0), dn_m)

    t011 = dot(wo011[...], pblk('tp_011', 0) * g['tp_011'], dn_m)
    psiV_parts = []
    for i in range(3):
        v1 = dot(wo101[...], pblk('tp_101', i) * g['tp_101'], dn_m)
        w = sum(pblk('tp_211', i * 3 + j) * rs_l[j] for j in range(3))
        v2 = dot(wo211[...], g['tp_211'] * w, dn_m)
        psiV_parts.append(t011 * rs_l[i] + v1 + v2)
    psiV = jnp.concatenate(psiV_parts, axis=0)

    D = []
    for i in range(3):
        d1 = dot(wo112[...], pblk('tp_112', i) * g['tp_112'], dn_m)
        w2 = sum(pblk('tp_222', i * 3 + j) * rs_l[j] for j in range(3))
        d2 = dot(wo222[...], g['tp_222'] * w2, dn_m)
        D.append(d1 + d2)
    psiD_parts = []
    for i in range(3):
        for j in range(3):
            e202 = dot(wo202[...], pblk('tp_202', i * 3 + j) * g['tp_202'], dn_m)
            psiD_parts.append(e202 + D[i] * rs_l[j])
    psiD = jnp.concatenate(psiD_parts, axis=0)

    psi = jnp.concatenate(
        [psiA, psiV, psiD, jnp.zeros((4, Bsz), f32)], axis=0)
    o_ref[...] = dot(psi, jnp.eye(FW, dtype=f32),
                     (((0,), (0,)), ((), ())))


def _tc_compute(G, Rpad, tcp, block_e):
    E = G.shape[0]
    grid = (E // block_e,)
    full = lambda a: pl.BlockSpec(a.shape, lambda i: (0,) * a.ndim)
    porder = ('wpT', 'wrT', 'woA', 'wo011', 'wo101', 'wo211', 'wo112',
              'wo202', 'wo222')
    return pl.pallas_call(
        _tc_body,
        grid=grid,
        in_specs=[
            pl.BlockSpec((block_e, FW), lambda i: (i, 0)),
            pl.BlockSpec((8, block_e), lambda i: (0, i)),
        ] + [full(tcp[k]) for k in porder],
        out_specs=pl.BlockSpec((block_e, FW), lambda i: (i, 0)),
        out_shape=jax.ShapeDtypeStruct((E, FW), jnp.float32),
    )(G, Rpad, *[tcp[k] for k in porder])


def _sc_gather(table, idx):
    info = plsc.get_sparse_core_info()
    nw = info.num_cores * info.num_subcores
    E = idx.shape[0]
    per_w = E // nw
    C = 1000
    mesh = plsc.VectorSubcoreMesh(core_axis_name="c", subcore_axis_name="s")

    @functools.partial(
        pl.kernel, mesh=mesh,
        compiler_params=pltpu.CompilerParams(use_tc_tiling_on_sc=False),
        out_type=jax.ShapeDtypeStruct((E, FW), jnp.float32),
        scratch_types=[
            pltpu.VMEM((C,), jnp.int32),
            pltpu.VMEM((C, FW), jnp.float32),
            pltpu.SemaphoreType.DMA,
        ],
    )
    def k(table_hbm, idx_hbm, out_hbm, idx_v, rows_v, sem):
        wid = lax.axis_index("s") * info.num_cores + lax.axis_index("c")
        base = wid * per_w

        def body(j, carry):
            o = base + j * C
            pltpu.sync_copy(idx_hbm.at[pl.ds(o, C)], idx_v)
            pltpu.async_copy(table_hbm.at[idx_v], rows_v, sem).wait()
            pltpu.sync_copy(rows_v, out_hbm.at[pl.ds(o, C)])
            return carry

        lax.fori_loop(0, per_w // C, body, 0)

    return k(table, idx)


def _sc_scatter_add(psi, src, n):
    info = plsc.get_sparse_core_info()
    ns = info.num_subcores
    E = psi.shape[0]
    per_w = E // ns
    C = 128
    NCH = E // C
    CZ = 1000
    NZ = n // CZ
    half = FW // 2
    mesh = plsc.VectorSubcoreMesh(core_axis_name="c", subcore_axis_name="s")

    @functools.partial(
        pl.kernel, mesh=mesh,
        compiler_params=pltpu.CompilerParams(use_tc_tiling_on_sc=False),
        out_type=jax.ShapeDtypeStruct((n, FW), jnp.float32),
        scratch_types=[
            pltpu.VMEM((C,), jnp.int32),
            pltpu.VMEM((C, half), jnp.float32),
            pltpu.VMEM_SHARED((n, half), jnp.float32),
        ],
    )
    def k(psi_hbm, idx_hbm, zero_hbm, out_hbm, idx_v, buf, acc):
        c = lax.axis_index("c")
        s = lax.axis_index("s")
        nz_own = (NZ + ns - 1) // ns

        def run(acc):
            def zbody(z, carry):
                zi = z * ns + s

                @pl.when(zi < NZ)
                def _():
                    pltpu.sync_copy(zero_hbm, acc.at[pl.ds(zi * CZ, CZ), :])
                return carry
            lax.fori_loop(0, nz_own, zbody, 0)
            plsc.subcore_barrier()

            def body(j, carry):
                k = j * ns + s

                @pl.when(k < NCH)
                def _():
                    o = k * C
                    pltpu.sync_copy(idx_hbm.at[pl.ds(o, C)], idx_v)
                    pltpu.sync_copy(
                        psi_hbm.at[pl.ds(o, C), pl.ds(c * half, half)], buf)
                    pltpu.sync_copy(buf, acc.at[idx_v], add=True)
                return carry
            lax.fori_loop(0, (NCH + ns - 1) // ns, body, 0)
            plsc.subcore_barrier()

            def wbody(z, carry):
                zi = z * ns + s

                @pl.when(zi < NZ)
                def _():
                    pltpu.sync_copy(
                        acc.at[pl.ds(zi * CZ, CZ), :],
                        out_hbm.at[pl.ds(zi * CZ, CZ), pl.ds(c * half, half)])
                return carry
            lax.fori_loop(0, nz_own, wbody, 0)

        run(acc)

    zero = jnp.zeros((CZ, half), jnp.float32)
    return k(psi, src, zero)


def _build_table(x_a, x_v, x_d):
    n = x_a.shape[0]
    xv = jnp.transpose(x_v, (0, 2, 1)).reshape(n, 24)
    xd = jnp.transpose(x_d, (0, 2, 3, 1)).reshape(n, 36)
    return jnp.concatenate(
        [x_a, xv, xd, jnp.zeros((n, 4), jnp.float32)], axis=1)


def _unpack_out(out):
    n = out.shape[0]
    B_a = out[:, 0:16]
    B_v = jnp.transpose(out[:, 16:40].reshape(n, 3, 8), (0, 2, 1))
    B_d = jnp.transpose(out[:, 40:76].reshape(n, 3, 3, 4), (0, 3, 1, 2))
    return B_a, B_v, B_d


def kernel(graph, r_ij, x_a, x_v, x_d, params):
    E = r_ij.shape[0]
    n = x_a.shape[0]
    src = graph[0]
    dst = graph[1]
    table = _build_table(x_a, x_v, x_d)
    Rpad = jnp.concatenate(
        [r_ij.T, jnp.zeros((5, E), jnp.float32)], axis=0)
    tcp = _build_tc_params(params)
    block_e = 3200 if E % 3200 == 0 else 16
    if E % 32000 == 0:
        G = _sc_gather(table, dst)
    else:
        G = jnp.take(table, dst, axis=0)
    psi = _tc_compute(G, Rpad, tcp, block_e)
    if E % 32000 == 0 and n % 1000 == 0:
        out = _sc_scatter_add(psi, src, n)
    else:
        out = jax.ops.segment_sum(psi, src, num_segments=n)
    return _unpack_out(out)
